# baseline (speedup 1.0000x reference)
import functools

import jax
import jax.numpy as jnp
from jax import lax
from jax.experimental import pallas as pl
from jax.experimental.pallas import tpu as pltpu
from jax.experimental.pallas import tpu_sc as plsc

_D = 64
_NC = 2
_NS = 16
_NW = _NC * _NS
_S = 50
_B = 4096
_L = 128
_DT = _D // 8


@functools.partial(jax.jit, static_argnames=("total",))
def _gather(weight, idx, total):
    del total
    mesh = plsc.VectorSubcoreMesh(core_axis_name="c", subcore_axis_name="s")
    b_per_w = _L * _S

    @functools.partial(
        pl.kernel,
        mesh=mesh,
        out_type=jax.ShapeDtypeStruct((_S, _DT, _B // _L, 8, _L), jnp.float32),
        scratch_types=[
            pltpu.VMEM((b_per_w,), jnp.int32),
            pltpu.VMEM((_S, _L), jnp.int32),
            pltpu.VMEM((2, _L, _D), jnp.float32),
            pltpu.VMEM((2, _DT, 8, _L), jnp.float32),
            pltpu.SemaphoreType.DMA,
            pltpu.SemaphoreType.DMA,
            pltpu.SemaphoreType.DMA,
            pltpu.SemaphoreType.DMA,
        ],
        compiler_params=pltpu.CompilerParams(
            use_tc_tiling_on_sc=False, needs_layout_passes=False),
    )
    def gather_kernel(table_hbm, idx_hbm, out5_hbm, idx_v, idxt_v, rows_v,
                      tile_v, sem_g0, sem_g1, sem_o0, sem_o1):
        bt = lax.axis_index("s") * _NC + lax.axis_index("c")
        iota = lax.iota(jnp.int32, 16)

        pltpu.sync_copy(idx_hbm.at[pl.ds(bt * b_per_w, b_per_w)], idx_v)

        def tr_idx(l0, carry):
            src = iota * _S + l0 * 16 * _S
            for s in range(_S):
                idxt_v[s, pl.ds(l0 * 16, 16)] = plsc.load_gather(
                    idx_v, [src + s])
            return carry

        lax.fori_loop(0, _L // 16, tr_idx, 0)

        def gather_desc(s, buf, sem):
            return pltpu.make_async_copy(
                table_hbm.at[idxt_v.at[s]], rows_v.at[buf], sem)

        def write_desc(s, buf, sem):
            return pltpu.make_async_copy(
                tile_v.at[buf], out5_hbm.at[s, pl.ds(0, _DT), bt], sem)

        def transpose_rows(buf):
            rows = rows_v.at[buf]

            def tr(l0, carry):
                rowidx = l0 * 16 + iota
                for dt in range(_DT):
                    for dr in range(8):
                        d = dt * 8 + dr
                        tile_v[buf, dt, dr, pl.ds(l0 * 16, 16)] = (
                            plsc.load_gather(
                                rows,
                                [rowidx, jnp.full((16,), d, jnp.int32)]))
                return carry

            lax.fori_loop(0, _L // 16, tr, 0)

        gather_desc(0, 0, sem_g0).start()
        gather_desc(1, 1, sem_g1).start()

        def half(k, s, buf, sem_g, sem_o):
            gather_desc(s, buf, sem_g).wait()

            @pl.when(k > 0)
            def _():
                write_desc(s, buf, sem_o).wait()

            transpose_rows(buf)
            write_desc(s, buf, sem_o).start()

            @pl.when(k < _S // 2 - 1)
            def _():
                gather_desc(s + 2, buf, sem_g).start()

        def body(k, carry):
            half(k, 2 * k, 0, sem_g0, sem_o0)
            half(k, 2 * k + 1, 1, sem_g1, sem_o1)
            return carry

        lax.fori_loop(0, _S // 2, body, 0)

        write_desc(_S - 2, 0, sem_o0).wait()
        write_desc(_S - 1, 1, sem_o1).wait()

    return gather_kernel(weight, idx)


def kernel(token_ids, weight):
    idx = token_ids.reshape(-1).astype(jnp.int32)
    out5 = _gather(weight, idx, idx.shape[0])
    return out5.transpose(2, 4, 0, 1, 3).reshape(_B, _S, _D)


# speedup vs baseline: 1.2983x; 1.2983x over previous
import functools

import jax
import jax.numpy as jnp
from jax import lax
from jax.experimental import pallas as pl
from jax.experimental.pallas import tpu as pltpu
from jax.experimental.pallas import tpu_sc as plsc

_D = 64
_NC = 2
_NS = 16
_NW = _NC * _NS
_S = 50
_B = 4096
_L = 128
_DT = _D // 8


@functools.partial(jax.jit, static_argnames=("total",))
def _gather(weight, idx, total):
    del total
    mesh = plsc.VectorSubcoreMesh(core_axis_name="c", subcore_axis_name="s")
    b_per_w = _L * _S

    @functools.partial(
        pl.kernel,
        mesh=mesh,
        out_type=jax.ShapeDtypeStruct((_S, _DT, _B // _L, 8, _L), jnp.float32),
        scratch_types=[
            pltpu.VMEM((b_per_w,), jnp.int32),
            pltpu.VMEM((_S, _L), jnp.int32),
            pltpu.VMEM((2, _L, _D), jnp.float32),
            pltpu.VMEM((2, _DT, 8, _L), jnp.float32),
            pltpu.SemaphoreType.DMA,
            pltpu.SemaphoreType.DMA,
            pltpu.SemaphoreType.DMA,
            pltpu.SemaphoreType.DMA,
        ],
        compiler_params=pltpu.CompilerParams(
            use_tc_tiling_on_sc=False, needs_layout_passes=False),
    )
    def gather_kernel(table_hbm, idx_hbm, out5_hbm, idx_v, idxt_v, rows_v,
                      tile_v, sem_g0, sem_g1, sem_o0, sem_o1):
        bt = lax.axis_index("s") * _NC + lax.axis_index("c")
        iota = lax.iota(jnp.int32, 16)

        pltpu.sync_copy(idx_hbm.at[pl.ds(bt * b_per_w, b_per_w)], idx_v)

        def tr_idx(l0, carry):
            src = iota * _S + l0 * 16 * _S
            for s in range(_S):
                idxt_v[s, pl.ds(l0 * 16, 16)] = plsc.load_gather(
                    idx_v, [src + s])
            return carry

        lax.fori_loop(0, _L // 16, tr_idx, 0)

        def gather_desc(s, buf, sem):
            return pltpu.make_async_copy(
                table_hbm.at[idxt_v.at[s]], rows_v.at[buf], sem)

        def write_desc(s, buf, sem):
            return pltpu.make_async_copy(
                tile_v.at[buf], out5_hbm.at[s, pl.ds(0, _DT), bt], sem)

        def transpose_rows(buf):
            rows = rows_v.at[buf]

            @plsc.parallel_loop(0, _L // 16, unroll=2)
            def tr(l0):
                rowidx = l0 * 16 + iota
                for dt in range(_DT):
                    for dr in range(8):
                        d = dt * 8 + dr
                        tile_v[buf, dt, dr, pl.ds(l0 * 16, 16)] = (
                            plsc.load_gather(
                                rows,
                                [rowidx, jnp.full((16,), d, jnp.int32)]))

        gather_desc(0, 0, sem_g0).start()
        gather_desc(1, 1, sem_g1).start()

        def half(k, s, buf, sem_g, sem_o):
            gather_desc(s, buf, sem_g).wait()

            @pl.when(k > 0)
            def _():
                write_desc(s, buf, sem_o).wait()

            transpose_rows(buf)
            write_desc(s, buf, sem_o).start()

            @pl.when(k < _S // 2 - 1)
            def _():
                gather_desc(s + 2, buf, sem_g).start()

        def body(k, carry):
            half(k, 2 * k, 0, sem_g0, sem_o0)
            half(k, 2 * k + 1, 1, sem_g1, sem_o1)
            return carry

        lax.fori_loop(0, _S // 2, body, 0)

        write_desc(_S - 2, 0, sem_o0).wait()
        write_desc(_S - 1, 1, sem_o1).wait()

    return gather_kernel(weight, idx)


def kernel(token_ids, weight):
    idx = token_ids.reshape(-1).astype(jnp.int32)
    out5 = _gather(weight, idx, idx.shape[0])
    return out5.transpose(2, 4, 0, 1, 3).reshape(_B, _S, _D)


# speedup vs baseline: 1.5093x; 1.1625x over previous
import functools

import jax
import jax.numpy as jnp
from jax import lax
from jax.experimental import pallas as pl
from jax.experimental.pallas import tpu as pltpu
from jax.experimental.pallas import tpu_sc as plsc

_D = 64
_NC = 2
_NS = 16
_NW = _NC * _NS
_S = 50
_B = 4096
_L = 128
_DT = _D // 8


@functools.partial(jax.jit, static_argnames=("total",))
def _gather(weight, idx, total):
    del total
    mesh = plsc.VectorSubcoreMesh(core_axis_name="c", subcore_axis_name="s")
    b_per_w = _L * _S

    @functools.partial(
        pl.kernel,
        mesh=mesh,
        out_type=jax.ShapeDtypeStruct((_S, _DT, _B // _L, 8, _L), jnp.float32),
        scratch_types=[
            pltpu.VMEM((b_per_w,), jnp.int32),
            pltpu.VMEM((_S, _L), jnp.int32),
            pltpu.VMEM((2, _L, _D), jnp.float32),
            pltpu.VMEM((2, _DT, 8, _L), jnp.float32),
            pltpu.SemaphoreType.DMA,
            pltpu.SemaphoreType.DMA,
            pltpu.SemaphoreType.DMA,
            pltpu.SemaphoreType.DMA,
        ],
        compiler_params=pltpu.CompilerParams(
            use_tc_tiling_on_sc=False, needs_layout_passes=False),
    )
    def gather_kernel(table_hbm, idx_hbm, out5_hbm, idx_v, idxt_v, rows_v,
                      tile_v, sem_g0, sem_g1, sem_o0, sem_o1):
        bt = lax.axis_index("s") * _NC + lax.axis_index("c")
        iota = lax.iota(jnp.int32, 16)

        pltpu.sync_copy(idx_hbm.at[pl.ds(bt * b_per_w, b_per_w)], idx_v)

        def tr_idx(l0, carry):
            src = iota * _S + l0 * 16 * _S
            for s in range(_S):
                idxt_v[s, pl.ds(l0 * 16, 16)] = plsc.load_gather(
                    idx_v, [src + s])
            return carry

        lax.fori_loop(0, _L // 16, tr_idx, 0)

        def gather_desc(s, buf, sem):
            return pltpu.make_async_copy(
                table_hbm.at[idxt_v.at[s]], rows_v.at[buf], sem)

        def write_desc(s, buf, sem):
            return pltpu.make_async_copy(
                tile_v.at[buf], out5_hbm.at[s, pl.ds(0, _DT), bt], sem)

        _dt_idx = [(q * 16 + iota) // 8 for q in range(_D // 16)]
        _dr_idx = [(q * 16 + iota) % 8 for q in range(_D // 16)]

        def transpose_rows(buf):
            rows = rows_v.at[buf]
            tile = tile_v.at[buf]

            @plsc.parallel_loop(0, _L, unroll=4)
            def tr(l):
                lane = jnp.broadcast_to(l, (16,))
                for q in range(_D // 16):
                    v = rows[l, pl.ds(q * 16, 16)]
                    plsc.store_scatter(tile, [_dt_idx[q], _dr_idx[q], lane],
                                       v)

        gather_desc(0, 0, sem_g0).start()
        gather_desc(1, 1, sem_g1).start()

        def half(k, s, buf, sem_g, sem_o):
            gather_desc(s, buf, sem_g).wait()

            @pl.when(k > 0)
            def _():
                write_desc(s, buf, sem_o).wait()

            transpose_rows(buf)
            write_desc(s, buf, sem_o).start()

            @pl.when(k < _S // 2 - 1)
            def _():
                gather_desc(s + 2, buf, sem_g).start()

        def body(k, carry):
            half(k, 2 * k, 0, sem_g0, sem_o0)
            half(k, 2 * k + 1, 1, sem_g1, sem_o1)
            return carry

        lax.fori_loop(0, _S // 2, body, 0)

        write_desc(_S - 2, 0, sem_o0).wait()
        write_desc(_S - 1, 1, sem_o1).wait()

    return gather_kernel(weight, idx)


def kernel(token_ids, weight):
    idx = token_ids.reshape(-1).astype(jnp.int32)
    out5 = _gather(weight, idx, idx.shape[0])
    return out5.transpose(2, 4, 0, 1, 3).reshape(_B, _S, _D)
